# baseline (speedup 1.0000x reference)
import functools

import jax
import jax.numpy as jnp
from jax import lax
from jax.experimental import pallas as pl
from jax.experimental.pallas import tpu as pltpu
from jax.experimental.pallas import tpu_sc as plsc

N = 10000
E = 160000
D = 128
NB = 20
C = 32
NCHUNK = 4

BN = 1000
BE2 = 2000
BE = 40
NTILES = 16
EPW = E // NTILES
NPW = 624
NTAIL = N - NPW * NTILES


def _stage_a_body(ns_ref, w1_ref, b1_ref, w2c_ref, b2c_ref, eqt_ref, t_ref):
    ns = ns_ref[...]
    h = ns @ w1_ref[...] + b1_ref[...][None, :]
    h = h * jax.nn.sigmoid(h)
    so = h @ w2c_ref[0] + b2c_ref[0]
    t_ref[0] = jnp.concatenate([so, eqt_ref[0]], axis=1)


def _stage_b_body(rbf_ref, fcut_ref, uv_ref, wrc_ref, brc_ref, f_ref):
    f = (rbf_ref[...] @ wrc_ref[0] + brc_ref[0]) * fcut_ref[...]
    f_m = f[:, 0:C]
    f_e = f[:, C:2 * C]
    f_s = f[:, 2 * C:3 * C]
    pad = jnp.zeros((f.shape[0], C - 3), jnp.float32)
    f_ref[0] = jnp.concatenate([f_m, f_s, f_e, uv_ref[...], pad], axis=1)


ESB = 2000
SBB = ESB // BE
NSB = EPW // ESB
SBPAIR = SBB // 2


def _sc_kernel(t_hbm, f_hbm, ctr_hbm, nbr_hbm, init_hbm, out_hbm,
               acc, nbr_sb, ctr_sb, tg_v, f_v, msg_v, gsem, fsem, ssem):
    cid = lax.axis_index("c")
    sid = lax.axis_index("s")
    nstart = sid * NPW

    def compute(slot):
        def edge(e, carry2):
            uvv = f_v[slot][e, pl.ds(3 * C, 16)]
            ub = [jnp.full((16,), uvv[comp], jnp.float32)
                  for comp in range(3)]
            for j in (0, 16):
                tm = tg_v[slot][e, pl.ds(0 + j, 16)]
                te = tg_v[slot][e, pl.ds(C + j, 16)]
                ts = tg_v[slot][e, pl.ds(2 * C + j, 16)]
                a = ts * f_v[slot][e, pl.ds(C + j, 16)]
                b = te * f_v[slot][e, pl.ds(2 * C + j, 16)]
                msg_v[e, pl.ds(0 + j, 16)] = tm * f_v[slot][e, pl.ds(0 + j, 16)]
                for comp in range(3):
                    tx = tg_v[slot][e, pl.ds((3 + comp) * C + j, 16)]
                    msg_v[e, pl.ds((1 + comp) * C + j, 16)] = tx * a + b * ub[comp]
            return carry2

        lax.fori_loop(0, BE, edge, 0)

    for chunk in range(NCHUNK):
        @pl.when(cid == chunk // 2)
        def _(chunk=chunk):
            pltpu.sync_copy(init_hbm.at[chunk, pl.ds(nstart, NPW)],
                            acc.at[pl.ds(nstart, NPW)])

            @pl.when(sid == NTILES - 1)
            def _():
                pltpu.sync_copy(init_hbm.at[chunk, pl.ds(NPW * NTILES, NTAIL)],
                                acc.at[pl.ds(NPW * NTILES, NTAIL)])
            plsc.subcore_barrier()

            def superblock(s, carry):
                row0 = sid * (EPW // BE) + s * SBB
                e0s = sid * EPW + s * ESB
                pltpu.sync_copy(nbr_hbm.at[pl.ds(row0, SBB)], nbr_sb)
                pltpu.sync_copy(ctr_hbm.at[pl.ds(row0, SBB)], ctr_sb)

                def load_and_gather(b, slot):
                    pltpu.async_copy(
                        f_hbm.at[chunk, pl.ds(e0s + b * BE, BE)],
                        f_v[slot], fsem[slot])
                    pltpu.async_copy(t_hbm.at[chunk].at[nbr_sb.at[b]],
                                     tg_v[slot], gsem[slot])

                def wait_loads(slot):
                    pltpu.make_async_copy(
                        f_hbm.at[chunk, pl.ds(0, BE)], f_v[slot],
                        fsem[slot]).wait()
                    pltpu.make_async_copy(t_hbm.at[chunk].at[nbr_sb.at[0]],
                                          tg_v[slot], gsem[slot]).wait()

                def do_scatter(b, slot):
                    pltpu.async_copy(msg_v, acc.at[ctr_sb.at[b]], ssem[slot],
                                     add=True)
                    pltpu.make_async_copy(msg_v, acc.at[ctr_sb.at[0]],
                                          ssem[slot]).wait()

                load_and_gather(0, 0)

                def pair(g, carry2):
                    load_and_gather(2 * g + 1, 1)
                    wait_loads(0)
                    compute(0)
                    do_scatter(2 * g, 0)

                    @pl.when(g < SBPAIR - 1)
                    def _():
                        load_and_gather(2 * g + 2, 0)
                    wait_loads(1)
                    compute(1)
                    do_scatter(2 * g + 1, 1)
                    return carry2

                lax.fori_loop(0, SBPAIR, pair, 0)
                return carry

            lax.fori_loop(0, NSB, superblock, 0)
            plsc.subcore_barrier()
            pltpu.sync_copy(acc.at[pl.ds(nstart, NPW)],
                            out_hbm.at[chunk, pl.ds(nstart, NPW)])

            @pl.when(sid == NTILES - 1)
            def _():
                pltpu.sync_copy(acc.at[pl.ds(NPW * NTILES, NTAIL)],
                                out_hbm.at[chunk, pl.ds(NPW * NTILES, NTAIL)])
            plsc.subcore_barrier()


def kernel(node_scalar, node_equi, rbf, fcut, uvec, edge_index,
           W1, b1, W2, b2, Wr, br):
    f32 = jnp.float32

    def chunk_cols(w):
        return jnp.stack([
            jnp.concatenate([w[..., k * D + c * C:k * D + c * C + C]
                             for k in range(3)], axis=-1)
            for c in range(NCHUNK)], axis=0)

    W2c = chunk_cols(W2)
    b2c = chunk_cols(b2)[:, None, :]
    Wrc = chunk_cols(Wr)
    brc = chunk_cols(br)[:, None, :]

    eqt = jnp.stack([node_equi[:, :, c * C:(c + 1) * C].reshape(N, 3 * C)
                     for c in range(NCHUNK)], axis=0)
    init = jnp.stack([
        jnp.concatenate([node_scalar[:, c * C:(c + 1) * C],
                         eqt[c]], axis=1)
        for c in range(NCHUNK)], axis=0)

    t_tab = pl.pallas_call(
        _stage_a_body,
        grid=(NCHUNK, N // BN),
        in_specs=[
            pl.BlockSpec((BN, D), lambda c, i: (i, 0)),
            pl.BlockSpec((D, D), lambda c, i: (0, 0)),
            pl.BlockSpec((D,), lambda c, i: (0,)),
            pl.BlockSpec((1, D, 3 * C), lambda c, i: (c, 0, 0)),
            pl.BlockSpec((1, 1, 3 * C), lambda c, i: (c, 0, 0)),
            pl.BlockSpec((1, BN, 3 * C), lambda c, i: (c, i, 0)),
        ],
        out_specs=pl.BlockSpec((1, BN, 6 * C), lambda c, i: (c, i, 0)),
        out_shape=jax.ShapeDtypeStruct((NCHUNK, N, 6 * C), f32),
    )(node_scalar, W1, b1, W2c, b2c, eqt)

    f_tab = pl.pallas_call(
        _stage_b_body,
        grid=(NCHUNK, E // BE2),
        in_specs=[
            pl.BlockSpec((BE2, NB), lambda c, i: (i, 0)),
            pl.BlockSpec((BE2, 1), lambda c, i: (i, 0)),
            pl.BlockSpec((BE2, 3), lambda c, i: (i, 0)),
            pl.BlockSpec((1, NB, 3 * C), lambda c, i: (c, 0, 0)),
            pl.BlockSpec((1, 1, 3 * C), lambda c, i: (c, 0, 0)),
        ],
        out_specs=pl.BlockSpec((1, BE2, 4 * C), lambda c, i: (c, i, 0)),
        out_shape=jax.ShapeDtypeStruct((NCHUNK, E, 4 * C), f32),
    )(rbf, fcut, uvec, Wrc, brc)

    mesh = plsc.VectorSubcoreMesh(core_axis_name="c", subcore_axis_name="s")
    sc = pl.kernel(
        _sc_kernel,
        out_type=jax.ShapeDtypeStruct((NCHUNK, N, 4 * C), f32),
        mesh=mesh,
        scratch_types=[
            pltpu.VMEM_SHARED((N, 4 * C), f32),
            pltpu.VMEM((SBB, BE), jnp.int32),
            pltpu.VMEM((SBB, BE), jnp.int32),
            [pltpu.VMEM((BE, 6 * C), f32) for _ in range(2)],
            [pltpu.VMEM((BE, 4 * C), f32) for _ in range(2)],
            pltpu.VMEM((BE, 4 * C), f32),
            [pltpu.SemaphoreType.DMA for _ in range(2)],
            [pltpu.SemaphoreType.DMA for _ in range(2)],
            [pltpu.SemaphoreType.DMA for _ in range(2)],
        ],
        compiler_params=pltpu.CompilerParams(use_tc_tiling_on_sc=False),
    )
    ctr2 = edge_index[0].reshape(E // BE, BE)
    nbr2 = edge_index[1].reshape(E // BE, BE)
    out = sc(t_tab, f_tab, ctr2, nbr2, init)

    new_scalar = jnp.moveaxis(out[:, :, 0:C], 0, 1).reshape(N, D)
    new_equi = jnp.transpose(out[:, :, C:].reshape(NCHUNK, N, 3, C),
                             (1, 2, 0, 3)).reshape(N, 3, D)
    return (new_scalar, new_equi)


# speedup vs baseline: 1.2867x; 1.2867x over previous
import functools

import jax
import jax.numpy as jnp
from jax import lax
from jax.experimental import pallas as pl
from jax.experimental.pallas import tpu as pltpu
from jax.experimental.pallas import tpu_sc as plsc

N = 10000
E = 160000
D = 128
NB = 20
C = 32
NCHUNK = 4

BN = 1000
BE2 = 2000
BE = 40
NTILES = 16
EPW = E // NTILES
NPW = 624
NTAIL = N - NPW * NTILES


def _stage_a_body(ns_ref, w1_ref, b1_ref, w2c_ref, b2c_ref, eqt_ref, t_ref):
    ns = ns_ref[...]
    h = ns @ w1_ref[...] + b1_ref[...][None, :]
    h = h * jax.nn.sigmoid(h)
    so = h @ w2c_ref[0] + b2c_ref[0]
    t_ref[0] = jnp.concatenate([so, eqt_ref[0]], axis=1)


def _stage_b_body(rbf_ref, fcut_ref, uv_ref, w24_ref, f_ref):
    fc = fcut_ref[...]
    x = jnp.concatenate([rbf_ref[...] * fc, fc, uv_ref[...]], axis=1)
    f_ref[0] = jnp.dot(x, w24_ref[0], precision=jax.lax.Precision.HIGHEST)


ESB = 2000
SBB = ESB // BE
NSB = EPW // ESB
SBPAIR = SBB // 2


def _sc_kernel(t_hbm, f_hbm, ctr_hbm, nbr_hbm, init_hbm, out_hbm,
               acc, nbr_sb, ctr_sb, tg_v, f_v, msg_v, gsem, fsem, ssem):
    cid = lax.axis_index("c")
    sid = lax.axis_index("s")
    nstart = sid * NPW

    def compute(slot):
        @plsc.parallel_loop(0, BE, 1, unroll=2)
        def edge(e):
            uvv = f_v[slot][e, pl.ds(3 * C, 16)]
            ub = [jnp.full((16,), uvv[comp], jnp.float32)
                  for comp in range(3)]
            for j in (0, 16):
                tm = tg_v[slot][e, pl.ds(0 + j, 16)]
                te = tg_v[slot][e, pl.ds(C + j, 16)]
                ts = tg_v[slot][e, pl.ds(2 * C + j, 16)]
                a = ts * f_v[slot][e, pl.ds(C + j, 16)]
                b = te * f_v[slot][e, pl.ds(2 * C + j, 16)]
                msg_v[e, pl.ds(0 + j, 16)] = tm * f_v[slot][e, pl.ds(0 + j, 16)]
                for comp in range(3):
                    tx = tg_v[slot][e, pl.ds((3 + comp) * C + j, 16)]
                    msg_v[e, pl.ds((1 + comp) * C + j, 16)] = tx * a + b * ub[comp]

    for chunk in range(NCHUNK):
        @pl.when(cid == chunk // 2)
        def _(chunk=chunk):
            pltpu.sync_copy(init_hbm.at[chunk, pl.ds(nstart, NPW)],
                            acc.at[pl.ds(nstart, NPW)])

            @pl.when(sid == NTILES - 1)
            def _():
                pltpu.sync_copy(init_hbm.at[chunk, pl.ds(NPW * NTILES, NTAIL)],
                                acc.at[pl.ds(NPW * NTILES, NTAIL)])
            plsc.subcore_barrier()

            def superblock(s, carry):
                row0 = sid * (EPW // BE) + s * SBB
                e0s = sid * EPW + s * ESB
                pltpu.sync_copy(nbr_hbm.at[pl.ds(row0, SBB)], nbr_sb)
                pltpu.sync_copy(ctr_hbm.at[pl.ds(row0, SBB)], ctr_sb)

                def load_and_gather(b, slot):
                    pltpu.async_copy(
                        f_hbm.at[chunk, pl.ds(e0s + b * BE, BE)],
                        f_v[slot], fsem[slot])
                    pltpu.async_copy(t_hbm.at[chunk].at[nbr_sb.at[b]],
                                     tg_v[slot], gsem[slot])

                def wait_loads(slot):
                    pltpu.make_async_copy(
                        f_hbm.at[chunk, pl.ds(0, BE)], f_v[slot],
                        fsem[slot]).wait()
                    pltpu.make_async_copy(t_hbm.at[chunk].at[nbr_sb.at[0]],
                                          tg_v[slot], gsem[slot]).wait()

                def do_scatter(b, slot):
                    pltpu.async_copy(msg_v, acc.at[ctr_sb.at[b]], ssem[slot],
                                     add=True)
                    pltpu.make_async_copy(msg_v, acc.at[ctr_sb.at[0]],
                                          ssem[slot]).wait()

                load_and_gather(0, 0)

                def pair(g, carry2):
                    load_and_gather(2 * g + 1, 1)
                    wait_loads(0)
                    compute(0)
                    do_scatter(2 * g, 0)

                    @pl.when(g < SBPAIR - 1)
                    def _():
                        load_and_gather(2 * g + 2, 0)
                    wait_loads(1)
                    compute(1)
                    do_scatter(2 * g + 1, 1)
                    return carry2

                lax.fori_loop(0, SBPAIR, pair, 0)
                return carry

            lax.fori_loop(0, NSB, superblock, 0)
            plsc.subcore_barrier()
            pltpu.sync_copy(acc.at[pl.ds(nstart, NPW)],
                            out_hbm.at[chunk, pl.ds(nstart, NPW)])

            @pl.when(sid == NTILES - 1)
            def _():
                pltpu.sync_copy(acc.at[pl.ds(NPW * NTILES, NTAIL)],
                                out_hbm.at[chunk, pl.ds(NPW * NTILES, NTAIL)])
            plsc.subcore_barrier()


def kernel(node_scalar, node_equi, rbf, fcut, uvec, edge_index,
           W1, b1, W2, b2, Wr, br):
    f32 = jnp.float32

    def chunk_cols(w):
        return jnp.stack([
            jnp.concatenate([w[..., k * D + c * C:k * D + c * C + C]
                             for k in range(3)], axis=-1)
            for c in range(NCHUNK)], axis=0)

    W2c = chunk_cols(W2)
    b2c = chunk_cols(b2)[:, None, :]
    Wrc = chunk_cols(Wr)
    brc = chunk_cols(br)
    perm = jnp.concatenate([jnp.arange(C), jnp.arange(2 * C, 3 * C),
                            jnp.arange(C, 2 * C)])
    W24 = jnp.zeros((NCHUNK, NB + 4, 4 * C), f32)
    W24 = W24.at[:, 0:NB, 0:3 * C].set(Wrc[:, :, perm])
    W24 = W24.at[:, NB, 0:3 * C].set(brc[:, perm])
    W24 = W24.at[:, NB + 1:NB + 4, 3 * C:3 * C + 3].set(
        jnp.broadcast_to(jnp.eye(3, dtype=f32), (NCHUNK, 3, 3)))

    eqt = jnp.stack([node_equi[:, :, c * C:(c + 1) * C].reshape(N, 3 * C)
                     for c in range(NCHUNK)], axis=0)
    init = jnp.stack([
        jnp.concatenate([node_scalar[:, c * C:(c + 1) * C],
                         eqt[c]], axis=1)
        for c in range(NCHUNK)], axis=0)

    t_tab = pl.pallas_call(
        _stage_a_body,
        grid=(NCHUNK, N // BN),
        in_specs=[
            pl.BlockSpec((BN, D), lambda c, i: (i, 0)),
            pl.BlockSpec((D, D), lambda c, i: (0, 0)),
            pl.BlockSpec((D,), lambda c, i: (0,)),
            pl.BlockSpec((1, D, 3 * C), lambda c, i: (c, 0, 0)),
            pl.BlockSpec((1, 1, 3 * C), lambda c, i: (c, 0, 0)),
            pl.BlockSpec((1, BN, 3 * C), lambda c, i: (c, i, 0)),
        ],
        out_specs=pl.BlockSpec((1, BN, 6 * C), lambda c, i: (c, i, 0)),
        out_shape=jax.ShapeDtypeStruct((NCHUNK, N, 6 * C), f32),
    )(node_scalar, W1, b1, W2c, b2c, eqt)

    f_tab = pl.pallas_call(
        _stage_b_body,
        grid=(NCHUNK, E // BE2),
        in_specs=[
            pl.BlockSpec((BE2, NB), lambda c, i: (i, 0)),
            pl.BlockSpec((BE2, 1), lambda c, i: (i, 0)),
            pl.BlockSpec((BE2, 3), lambda c, i: (i, 0)),
            pl.BlockSpec((1, NB + 4, 4 * C), lambda c, i: (c, 0, 0)),
        ],
        out_specs=pl.BlockSpec((1, BE2, 4 * C), lambda c, i: (c, i, 0)),
        out_shape=jax.ShapeDtypeStruct((NCHUNK, E, 4 * C), f32),
    )(rbf, fcut, uvec, W24)

    mesh = plsc.VectorSubcoreMesh(core_axis_name="c", subcore_axis_name="s")
    sc = pl.kernel(
        _sc_kernel,
        out_type=jax.ShapeDtypeStruct((NCHUNK, N, 4 * C), f32),
        mesh=mesh,
        scratch_types=[
            pltpu.VMEM_SHARED((N, 4 * C), f32),
            pltpu.VMEM((SBB, BE), jnp.int32),
            pltpu.VMEM((SBB, BE), jnp.int32),
            [pltpu.VMEM((BE, 6 * C), f32) for _ in range(2)],
            [pltpu.VMEM((BE, 4 * C), f32) for _ in range(2)],
            pltpu.VMEM((BE, 4 * C), f32),
            [pltpu.SemaphoreType.DMA for _ in range(2)],
            [pltpu.SemaphoreType.DMA for _ in range(2)],
            [pltpu.SemaphoreType.DMA for _ in range(2)],
        ],
        compiler_params=pltpu.CompilerParams(use_tc_tiling_on_sc=False),
    )
    ctr2 = edge_index[0].reshape(E // BE, BE)
    nbr2 = edge_index[1].reshape(E // BE, BE)
    out = sc(t_tab, f_tab, ctr2, nbr2, init)

    new_scalar = jnp.moveaxis(out[:, :, 0:C], 0, 1).reshape(N, D)
    new_equi = jnp.transpose(out[:, :, C:].reshape(NCHUNK, N, 3, C),
                             (1, 2, 0, 3)).reshape(N, 3, D)
    return (new_scalar, new_equi)


# speedup vs baseline: 1.3333x; 1.0363x over previous
import functools

import jax
import jax.numpy as jnp
from jax import lax
from jax.experimental import pallas as pl
from jax.experimental.pallas import tpu as pltpu
from jax.experimental.pallas import tpu_sc as plsc

N = 10000
E = 160000
D = 128
NB = 20
C = 32
NCHUNK = 4

BN = 1000
BE2 = 2000
BE = 40
NTILES = 16
EPW = E // NTILES
NPW = 624
NTAIL = N - NPW * NTILES


def _stage_a_body(ns_ref, w1_ref, b1_ref, w2c_ref, b2c_ref, eqt_ref, t_ref):
    ns = ns_ref[...]
    h = ns @ w1_ref[...] + b1_ref[...][None, :]
    h = h * jax.nn.sigmoid(h)
    so = h @ w2c_ref[0] + b2c_ref[0]
    t_ref[0] = jnp.concatenate([so, eqt_ref[0]], axis=1)


def _stage_b_body(rbf_ref, fcut_ref, uv_ref, w20_ref, brow_ref, w3_ref, f_ref):
    fc = fcut_ref[...]
    f_ref[0] = ((rbf_ref[...] * fc) @ w20_ref[0]
                + fc * brow_ref[0]
                + jnp.dot(uv_ref[...], w3_ref[0],
                          precision=jax.lax.Precision.HIGHEST))


ESB = 2000
SBB = ESB // BE
NSB = EPW // ESB
SBPAIR = SBB // 2


def _sc_kernel(t_hbm, f_hbm, ctr_hbm, nbr_hbm, init_hbm, out_hbm,
               acc, nbr_sb, ctr_sb, tg_v, f_v, msg_v, gsem, fsem, ssem):
    cid = lax.axis_index("c")
    sid = lax.axis_index("s")
    nstart = sid * NPW

    def compute(slot):
        @plsc.parallel_loop(0, BE, 1, unroll=2)
        def edge(e):
            uvv = f_v[slot][e, pl.ds(3 * C, 16)]
            ub = [jnp.full((16,), uvv[comp], jnp.float32)
                  for comp in range(3)]
            for j in (0, 16):
                tm = tg_v[slot][e, pl.ds(0 + j, 16)]
                te = tg_v[slot][e, pl.ds(C + j, 16)]
                ts = tg_v[slot][e, pl.ds(2 * C + j, 16)]
                a = ts * f_v[slot][e, pl.ds(C + j, 16)]
                b = te * f_v[slot][e, pl.ds(2 * C + j, 16)]
                msg_v[e, pl.ds(0 + j, 16)] = tm * f_v[slot][e, pl.ds(0 + j, 16)]
                for comp in range(3):
                    tx = tg_v[slot][e, pl.ds((3 + comp) * C + j, 16)]
                    msg_v[e, pl.ds((1 + comp) * C + j, 16)] = tx * a + b * ub[comp]

    for chunk in range(NCHUNK):
        @pl.when(cid == chunk // 2)
        def _(chunk=chunk):
            pltpu.sync_copy(init_hbm.at[chunk, pl.ds(nstart, NPW)],
                            acc.at[pl.ds(nstart, NPW)])

            @pl.when(sid == NTILES - 1)
            def _():
                pltpu.sync_copy(init_hbm.at[chunk, pl.ds(NPW * NTILES, NTAIL)],
                                acc.at[pl.ds(NPW * NTILES, NTAIL)])
            plsc.subcore_barrier()

            def superblock(s, carry):
                row0 = sid * (EPW // BE) + s * SBB
                e0s = sid * EPW + s * ESB
                pltpu.sync_copy(nbr_hbm.at[pl.ds(row0, SBB)], nbr_sb)
                pltpu.sync_copy(ctr_hbm.at[pl.ds(row0, SBB)], ctr_sb)

                def load_and_gather(b, slot):
                    pltpu.async_copy(
                        f_hbm.at[chunk, pl.ds(e0s + b * BE, BE)],
                        f_v[slot], fsem[slot])
                    pltpu.async_copy(t_hbm.at[chunk].at[nbr_sb.at[b]],
                                     tg_v[slot], gsem[slot])

                def wait_loads(slot):
                    pltpu.make_async_copy(
                        f_hbm.at[chunk, pl.ds(0, BE)], f_v[slot],
                        fsem[slot]).wait()
                    pltpu.make_async_copy(t_hbm.at[chunk].at[nbr_sb.at[0]],
                                          tg_v[slot], gsem[slot]).wait()

                def do_scatter(b, slot):
                    pltpu.async_copy(msg_v, acc.at[ctr_sb.at[b]], ssem[slot],
                                     add=True)
                    pltpu.make_async_copy(msg_v, acc.at[ctr_sb.at[0]],
                                          ssem[slot]).wait()

                load_and_gather(0, 0)

                def pair(g, carry2):
                    load_and_gather(2 * g + 1, 1)
                    wait_loads(0)
                    compute(0)
                    do_scatter(2 * g, 0)

                    @pl.when(g < SBPAIR - 1)
                    def _():
                        load_and_gather(2 * g + 2, 0)
                    wait_loads(1)
                    compute(1)
                    do_scatter(2 * g + 1, 1)
                    return carry2

                lax.fori_loop(0, SBPAIR, pair, 0)
                return carry

            lax.fori_loop(0, NSB, superblock, 0)
            plsc.subcore_barrier()
            pltpu.sync_copy(acc.at[pl.ds(nstart, NPW)],
                            out_hbm.at[chunk, pl.ds(nstart, NPW)])

            @pl.when(sid == NTILES - 1)
            def _():
                pltpu.sync_copy(acc.at[pl.ds(NPW * NTILES, NTAIL)],
                                out_hbm.at[chunk, pl.ds(NPW * NTILES, NTAIL)])
            plsc.subcore_barrier()


def kernel(node_scalar, node_equi, rbf, fcut, uvec, edge_index,
           W1, b1, W2, b2, Wr, br):
    f32 = jnp.float32

    def chunk_cols(w):
        return jnp.stack([
            jnp.concatenate([w[..., k * D + c * C:k * D + c * C + C]
                             for k in range(3)], axis=-1)
            for c in range(NCHUNK)], axis=0)

    W2c = chunk_cols(W2)
    b2c = chunk_cols(b2)[:, None, :]
    Wrc = chunk_cols(Wr)
    brc = chunk_cols(br)
    perm = jnp.concatenate([jnp.arange(C), jnp.arange(2 * C, 3 * C),
                            jnp.arange(C, 2 * C)])
    W20 = jnp.zeros((NCHUNK, NB, 4 * C), f32).at[:, :, 0:3 * C].set(
        Wrc[:, :, perm])
    brow = jnp.zeros((NCHUNK, 1, 4 * C), f32).at[:, 0, 0:3 * C].set(
        brc[:, perm])
    W3 = jnp.zeros((NCHUNK, 3, 4 * C), f32).at[:, :, 3 * C:3 * C + 3].set(
        jnp.broadcast_to(jnp.eye(3, dtype=f32), (NCHUNK, 3, 3)))

    eqt = jnp.stack([node_equi[:, :, c * C:(c + 1) * C].reshape(N, 3 * C)
                     for c in range(NCHUNK)], axis=0)
    init = jnp.stack([
        jnp.concatenate([node_scalar[:, c * C:(c + 1) * C],
                         eqt[c]], axis=1)
        for c in range(NCHUNK)], axis=0)

    t_tab = pl.pallas_call(
        _stage_a_body,
        grid=(NCHUNK, N // BN),
        in_specs=[
            pl.BlockSpec((BN, D), lambda c, i: (i, 0)),
            pl.BlockSpec((D, D), lambda c, i: (0, 0)),
            pl.BlockSpec((D,), lambda c, i: (0,)),
            pl.BlockSpec((1, D, 3 * C), lambda c, i: (c, 0, 0)),
            pl.BlockSpec((1, 1, 3 * C), lambda c, i: (c, 0, 0)),
            pl.BlockSpec((1, BN, 3 * C), lambda c, i: (c, i, 0)),
        ],
        out_specs=pl.BlockSpec((1, BN, 6 * C), lambda c, i: (c, i, 0)),
        out_shape=jax.ShapeDtypeStruct((NCHUNK, N, 6 * C), f32),
    )(node_scalar, W1, b1, W2c, b2c, eqt)

    f_tab = pl.pallas_call(
        _stage_b_body,
        grid=(NCHUNK, E // BE2),
        in_specs=[
            pl.BlockSpec((BE2, NB), lambda c, i: (i, 0)),
            pl.BlockSpec((BE2, 1), lambda c, i: (i, 0)),
            pl.BlockSpec((BE2, 3), lambda c, i: (i, 0)),
            pl.BlockSpec((1, NB, 4 * C), lambda c, i: (c, 0, 0)),
            pl.BlockSpec((1, 1, 4 * C), lambda c, i: (c, 0, 0)),
            pl.BlockSpec((1, 3, 4 * C), lambda c, i: (c, 0, 0)),
        ],
        out_specs=pl.BlockSpec((1, BE2, 4 * C), lambda c, i: (c, i, 0)),
        out_shape=jax.ShapeDtypeStruct((NCHUNK, E, 4 * C), f32),
    )(rbf, fcut, uvec, W20, brow, W3)

    mesh = plsc.VectorSubcoreMesh(core_axis_name="c", subcore_axis_name="s")
    sc = pl.kernel(
        _sc_kernel,
        out_type=jax.ShapeDtypeStruct((NCHUNK, N, 4 * C), f32),
        mesh=mesh,
        scratch_types=[
            pltpu.VMEM_SHARED((N, 4 * C), f32),
            pltpu.VMEM((SBB, BE), jnp.int32),
            pltpu.VMEM((SBB, BE), jnp.int32),
            [pltpu.VMEM((BE, 6 * C), f32) for _ in range(2)],
            [pltpu.VMEM((BE, 4 * C), f32) for _ in range(2)],
            pltpu.VMEM((BE, 4 * C), f32),
            [pltpu.SemaphoreType.DMA for _ in range(2)],
            [pltpu.SemaphoreType.DMA for _ in range(2)],
            [pltpu.SemaphoreType.DMA for _ in range(2)],
        ],
        compiler_params=pltpu.CompilerParams(use_tc_tiling_on_sc=False),
    )
    ctr2 = edge_index[0].reshape(E // BE, BE)
    nbr2 = edge_index[1].reshape(E // BE, BE)
    out = sc(t_tab, f_tab, ctr2, nbr2, init)

    new_scalar = jnp.moveaxis(out[:, :, 0:C], 0, 1).reshape(N, D)
    new_equi = jnp.transpose(out[:, :, C:].reshape(NCHUNK, N, 3, C),
                             (1, 2, 0, 3)).reshape(N, 3, D)
    return (new_scalar, new_equi)


# speedup vs baseline: 1.5725x; 1.1794x over previous
import functools

import jax
import jax.numpy as jnp
from jax import lax
from jax.experimental import pallas as pl
from jax.experimental.pallas import tpu as pltpu
from jax.experimental.pallas import tpu_sc as plsc

N = 10000
E = 160000
D = 128
NB = 20
C = 32
NCHUNK = 4

BN = 1000
BE2 = 2000
BE = 40
NTILES = 16
EPW = E // NTILES
NPW = 624
NTAIL = N - NPW * NTILES


def _stage_a_body(ns_ref, w1_ref, b1_ref, w2c_ref, b2c_ref, eqt_ref, t_ref):
    ns = ns_ref[...]
    h = ns @ w1_ref[...] + b1_ref[...][None, :]
    h = h * jax.nn.sigmoid(h)
    so = h @ w2c_ref[0] + b2c_ref[0]
    t_ref[0] = jnp.concatenate([so, eqt_ref[0]], axis=1)


def _stage_b_body(rbf_ref, fcut_ref, uv_ref, w20_ref, brow_ref, w3_ref, f_ref):
    fc = fcut_ref[...]
    out = ((rbf_ref[...] * fc) @ w20_ref[...]
           + fc * brow_ref[...]
           + jnp.dot(uv_ref[...], w3_ref[...],
                     precision=jax.lax.Precision.HIGHEST))
    f_ref[...] = out.reshape(out.shape[0], NCHUNK, 4 * C)


ESB = 2000
SBB = ESB // BE
NSB = EPW // ESB
SBPAIR = SBB // 2


def _sc_kernel(t_hbm, f_hbm, ctr_hbm, nbr_hbm, init_hbm, out_hbm,
               acc, nbr_sb, ctr_sb, tg_v, f_v, msg_v, gsem, fsem, ssem):
    cid = lax.axis_index("c")
    sid = lax.axis_index("s")
    nstart = sid * NPW

    def compute(slot):
        @plsc.parallel_loop(0, BE, 1, unroll=2)
        def edge(e):
            uvv = f_v[slot][e, pl.ds(3 * C, 16)]
            ub = [jnp.full((16,), uvv[comp], jnp.float32)
                  for comp in range(3)]
            for j in (0, 16):
                tm = tg_v[slot][e, pl.ds(0 + j, 16)]
                te = tg_v[slot][e, pl.ds(C + j, 16)]
                ts = tg_v[slot][e, pl.ds(2 * C + j, 16)]
                a = ts * f_v[slot][e, pl.ds(C + j, 16)]
                b = te * f_v[slot][e, pl.ds(2 * C + j, 16)]
                msg_v[e, pl.ds(0 + j, 16)] = tm * f_v[slot][e, pl.ds(0 + j, 16)]
                for comp in range(3):
                    tx = tg_v[slot][e, pl.ds((3 + comp) * C + j, 16)]
                    msg_v[e, pl.ds((1 + comp) * C + j, 16)] = tx * a + b * ub[comp]

    for chunk in range(NCHUNK):
        @pl.when(cid == chunk // 2)
        def _(chunk=chunk):
            pltpu.sync_copy(init_hbm.at[chunk, pl.ds(nstart, NPW)],
                            acc.at[pl.ds(nstart, NPW)])

            @pl.when(sid == NTILES - 1)
            def _():
                pltpu.sync_copy(init_hbm.at[chunk, pl.ds(NPW * NTILES, NTAIL)],
                                acc.at[pl.ds(NPW * NTILES, NTAIL)])
            plsc.subcore_barrier()

            def superblock(s, carry):
                row0 = sid * (EPW // BE) + s * SBB
                e0s = sid * EPW + s * ESB
                pltpu.sync_copy(nbr_hbm.at[pl.ds(row0, SBB)], nbr_sb)
                pltpu.sync_copy(ctr_hbm.at[pl.ds(row0, SBB)], ctr_sb)

                def load_and_gather(b, slot):
                    pltpu.async_copy(
                        f_hbm.at[pl.ds(e0s + b * BE, BE), chunk],
                        f_v[slot], fsem[slot])
                    pltpu.async_copy(t_hbm.at[chunk].at[nbr_sb.at[b]],
                                     tg_v[slot], gsem[slot])

                def wait_loads(slot):
                    pltpu.make_async_copy(
                        f_hbm.at[pl.ds(0, BE), chunk], f_v[slot],
                        fsem[slot]).wait()
                    pltpu.make_async_copy(t_hbm.at[chunk].at[nbr_sb.at[0]],
                                          tg_v[slot], gsem[slot]).wait()

                def do_scatter(b, slot):
                    pltpu.async_copy(msg_v, acc.at[ctr_sb.at[b]], ssem[slot],
                                     add=True)
                    pltpu.make_async_copy(msg_v, acc.at[ctr_sb.at[0]],
                                          ssem[slot]).wait()

                load_and_gather(0, 0)

                def pair(g, carry2):
                    load_and_gather(2 * g + 1, 1)
                    wait_loads(0)
                    compute(0)
                    do_scatter(2 * g, 0)

                    @pl.when(g < SBPAIR - 1)
                    def _():
                        load_and_gather(2 * g + 2, 0)
                    wait_loads(1)
                    compute(1)
                    do_scatter(2 * g + 1, 1)
                    return carry2

                lax.fori_loop(0, SBPAIR, pair, 0)
                return carry

            lax.fori_loop(0, NSB, superblock, 0)
            plsc.subcore_barrier()
            pltpu.sync_copy(acc.at[pl.ds(nstart, NPW)],
                            out_hbm.at[chunk, pl.ds(nstart, NPW)])

            @pl.when(sid == NTILES - 1)
            def _():
                pltpu.sync_copy(acc.at[pl.ds(NPW * NTILES, NTAIL)],
                                out_hbm.at[chunk, pl.ds(NPW * NTILES, NTAIL)])
            plsc.subcore_barrier()


def kernel(node_scalar, node_equi, rbf, fcut, uvec, edge_index,
           W1, b1, W2, b2, Wr, br):
    f32 = jnp.float32

    def chunk_cols(w):
        return jnp.stack([
            jnp.concatenate([w[..., k * D + c * C:k * D + c * C + C]
                             for k in range(3)], axis=-1)
            for c in range(NCHUNK)], axis=0)

    W2c = chunk_cols(W2)
    b2c = chunk_cols(b2)[:, None, :]
    Wrc = chunk_cols(Wr)
    brc = chunk_cols(br)
    perm = jnp.concatenate([jnp.arange(C), jnp.arange(2 * C, 3 * C),
                            jnp.arange(C, 2 * C)])
    W20 = jnp.zeros((NCHUNK, NB, 4 * C), f32).at[:, :, 0:3 * C].set(
        Wrc[:, :, perm])
    brow = jnp.zeros((NCHUNK, 1, 4 * C), f32).at[:, 0, 0:3 * C].set(
        brc[:, perm])
    W3 = jnp.zeros((NCHUNK, 3, 4 * C), f32).at[:, :, 3 * C:3 * C + 3].set(
        jnp.broadcast_to(jnp.eye(3, dtype=f32), (NCHUNK, 3, 3)))
    W20f = jnp.moveaxis(W20, 0, 1).reshape(NB, NCHUNK * 4 * C)
    browf = jnp.moveaxis(brow, 0, 1).reshape(1, NCHUNK * 4 * C)
    W3f = jnp.moveaxis(W3, 0, 1).reshape(3, NCHUNK * 4 * C)

    eqt = jnp.stack([node_equi[:, :, c * C:(c + 1) * C].reshape(N, 3 * C)
                     for c in range(NCHUNK)], axis=0)
    init = jnp.stack([
        jnp.concatenate([node_scalar[:, c * C:(c + 1) * C],
                         eqt[c]], axis=1)
        for c in range(NCHUNK)], axis=0)

    t_tab = pl.pallas_call(
        _stage_a_body,
        grid=(NCHUNK, N // BN),
        in_specs=[
            pl.BlockSpec((BN, D), lambda c, i: (i, 0)),
            pl.BlockSpec((D, D), lambda c, i: (0, 0)),
            pl.BlockSpec((D,), lambda c, i: (0,)),
            pl.BlockSpec((1, D, 3 * C), lambda c, i: (c, 0, 0)),
            pl.BlockSpec((1, 1, 3 * C), lambda c, i: (c, 0, 0)),
            pl.BlockSpec((1, BN, 3 * C), lambda c, i: (c, i, 0)),
        ],
        out_specs=pl.BlockSpec((1, BN, 6 * C), lambda c, i: (c, i, 0)),
        out_shape=jax.ShapeDtypeStruct((NCHUNK, N, 6 * C), f32),
    )(node_scalar, W1, b1, W2c, b2c, eqt)

    f_tab = pl.pallas_call(
        _stage_b_body,
        grid=(E // BE2,),
        in_specs=[
            pl.BlockSpec((BE2, NB), lambda i: (i, 0)),
            pl.BlockSpec((BE2, 1), lambda i: (i, 0)),
            pl.BlockSpec((BE2, 3), lambda i: (i, 0)),
            pl.BlockSpec((NB, NCHUNK * 4 * C), lambda i: (0, 0)),
            pl.BlockSpec((1, NCHUNK * 4 * C), lambda i: (0, 0)),
            pl.BlockSpec((3, NCHUNK * 4 * C), lambda i: (0, 0)),
        ],
        out_specs=pl.BlockSpec((BE2, NCHUNK, 4 * C), lambda i: (i, 0, 0)),
        out_shape=jax.ShapeDtypeStruct((E, NCHUNK, 4 * C), f32),
    )(rbf, fcut, uvec, W20f, browf, W3f)

    mesh = plsc.VectorSubcoreMesh(core_axis_name="c", subcore_axis_name="s")
    sc = pl.kernel(
        _sc_kernel,
        out_type=jax.ShapeDtypeStruct((NCHUNK, N, 4 * C), f32),
        mesh=mesh,
        scratch_types=[
            pltpu.VMEM_SHARED((N, 4 * C), f32),
            pltpu.VMEM((SBB, BE), jnp.int32),
            pltpu.VMEM((SBB, BE), jnp.int32),
            [pltpu.VMEM((BE, 6 * C), f32) for _ in range(2)],
            [pltpu.VMEM((BE, 4 * C), f32) for _ in range(2)],
            pltpu.VMEM((BE, 4 * C), f32),
            [pltpu.SemaphoreType.DMA for _ in range(2)],
            [pltpu.SemaphoreType.DMA for _ in range(2)],
            [pltpu.SemaphoreType.DMA for _ in range(2)],
        ],
        compiler_params=pltpu.CompilerParams(use_tc_tiling_on_sc=False),
    )
    ctr2 = edge_index[0].reshape(E // BE, BE)
    nbr2 = edge_index[1].reshape(E // BE, BE)
    out = sc(t_tab, f_tab, ctr2, nbr2, init)

    new_scalar = jnp.moveaxis(out[:, :, 0:C], 0, 1).reshape(N, D)
    new_equi = jnp.transpose(out[:, :, C:].reshape(NCHUNK, N, 3, C),
                             (1, 2, 0, 3)).reshape(N, 3, D)
    return (new_scalar, new_equi)


# speedup vs baseline: 1.6801x; 1.0684x over previous
import functools

import jax
import jax.numpy as jnp
from jax import lax
from jax.experimental import pallas as pl
from jax.experimental.pallas import tpu as pltpu
from jax.experimental.pallas import tpu_sc as plsc

N = 10000
E = 160000
D = 128
NB = 20
C = 32
NCHUNK = 4

BN = 1000
BE2 = 2000
BE = 40
NTILES = 16
EPW = E // NTILES
NPW = 624
NTAIL = N - NPW * NTILES


def _stage_a_body(ns_ref, w1_ref, b1_ref, w2c_ref, b2c_ref, eqt_ref, t_ref):
    ns = ns_ref[...]
    h = ns @ w1_ref[...] + b1_ref[...][None, :]
    h = h * jax.nn.sigmoid(h)
    so = h @ w2c_ref[0] + b2c_ref[0]
    t_ref[0] = jnp.concatenate([so, eqt_ref[0]], axis=1)


def _stage_b_body(rbf_ref, fcut_ref, uv_ref, w20_ref, brow_ref, w3_ref, f_ref):
    fc = fcut_ref[...]
    out = ((rbf_ref[...] * fc) @ w20_ref[...]
           + fc * brow_ref[...]
           + jnp.dot(uv_ref[...], w3_ref[...],
                     precision=jax.lax.Precision.HIGHEST))
    f_ref[...] = out.reshape(out.shape[0], f_ref.shape[1], 4 * C)


ESB = 2000
SBB = ESB // BE
NSB = EPW // ESB
SBPAIR = SBB // 2


def _sc_kernel(base, t_hbm, f_hbm, ctr_hbm, nbr_hbm, init_hbm, out_hbm,
               acc, nbr_sb, ctr_sb, tg_v, f_v, msg_v, gsem, fsem, ssem):
    cid = lax.axis_index("c")
    sid = lax.axis_index("s")
    nstart = sid * NPW

    def compute(slot):
        @plsc.parallel_loop(0, BE, 1, unroll=2)
        def edge(e):
            uvv = f_v[slot][e, pl.ds(3 * C, 16)]
            ub = [jnp.full((16,), uvv[comp], jnp.float32)
                  for comp in range(3)]
            for j in (0, 16):
                tm = tg_v[slot][e, pl.ds(0 + j, 16)]
                te = tg_v[slot][e, pl.ds(C + j, 16)]
                ts = tg_v[slot][e, pl.ds(2 * C + j, 16)]
                a = ts * f_v[slot][e, pl.ds(C + j, 16)]
                b = te * f_v[slot][e, pl.ds(2 * C + j, 16)]
                msg_v[e, pl.ds(0 + j, 16)] = tm * f_v[slot][e, pl.ds(0 + j, 16)]
                for comp in range(3):
                    tx = tg_v[slot][e, pl.ds((3 + comp) * C + j, 16)]
                    msg_v[e, pl.ds((1 + comp) * C + j, 16)] = tx * a + b * ub[comp]

    for local in range(2):
        chunk = base + local
        @pl.when(cid == local)
        def _(chunk=chunk, local=local):
            pltpu.sync_copy(init_hbm.at[local, pl.ds(nstart, NPW)],
                            acc.at[pl.ds(nstart, NPW)])

            @pl.when(sid == NTILES - 1)
            def _():
                pltpu.sync_copy(init_hbm.at[local, pl.ds(NPW * NTILES, NTAIL)],
                                acc.at[pl.ds(NPW * NTILES, NTAIL)])
            plsc.subcore_barrier()

            def superblock(s, carry):
                row0 = sid * (EPW // BE) + s * SBB
                e0s = sid * EPW + s * ESB
                pltpu.sync_copy(nbr_hbm.at[pl.ds(row0, SBB)], nbr_sb)
                pltpu.sync_copy(ctr_hbm.at[pl.ds(row0, SBB)], ctr_sb)

                def load_and_gather(b, slot):
                    pltpu.async_copy(
                        f_hbm.at[pl.ds(e0s + b * BE, BE), local],
                        f_v[slot], fsem[slot])
                    pltpu.async_copy(t_hbm.at[chunk].at[nbr_sb.at[b]],
                                     tg_v[slot], gsem[slot])

                def wait_loads(slot):
                    pltpu.make_async_copy(
                        f_hbm.at[pl.ds(0, BE), local], f_v[slot],
                        fsem[slot]).wait()
                    pltpu.make_async_copy(t_hbm.at[chunk].at[nbr_sb.at[0]],
                                          tg_v[slot], gsem[slot]).wait()

                def do_scatter(b, slot):
                    pltpu.async_copy(msg_v, acc.at[ctr_sb.at[b]], ssem[slot],
                                     add=True)
                    pltpu.make_async_copy(msg_v, acc.at[ctr_sb.at[0]],
                                          ssem[slot]).wait()

                load_and_gather(0, 0)

                def pair(g, carry2):
                    load_and_gather(2 * g + 1, 1)
                    wait_loads(0)
                    compute(0)
                    do_scatter(2 * g, 0)

                    @pl.when(g < SBPAIR - 1)
                    def _():
                        load_and_gather(2 * g + 2, 0)
                    wait_loads(1)
                    compute(1)
                    do_scatter(2 * g + 1, 1)
                    return carry2

                lax.fori_loop(0, SBPAIR, pair, 0)
                return carry

            lax.fori_loop(0, NSB, superblock, 0)
            plsc.subcore_barrier()
            pltpu.sync_copy(acc.at[pl.ds(nstart, NPW)],
                            out_hbm.at[local, pl.ds(nstart, NPW)])

            @pl.when(sid == NTILES - 1)
            def _():
                pltpu.sync_copy(acc.at[pl.ds(NPW * NTILES, NTAIL)],
                                out_hbm.at[local, pl.ds(NPW * NTILES, NTAIL)])
            plsc.subcore_barrier()


def kernel(node_scalar, node_equi, rbf, fcut, uvec, edge_index,
           W1, b1, W2, b2, Wr, br):
    f32 = jnp.float32

    def chunk_cols(w):
        return jnp.stack([
            jnp.concatenate([w[..., k * D + c * C:k * D + c * C + C]
                             for k in range(3)], axis=-1)
            for c in range(NCHUNK)], axis=0)

    W2c = chunk_cols(W2)
    b2c = chunk_cols(b2)[:, None, :]
    Wrc = chunk_cols(Wr)
    brc = chunk_cols(br)
    perm = jnp.concatenate([jnp.arange(C), jnp.arange(2 * C, 3 * C),
                            jnp.arange(C, 2 * C)])
    W20 = jnp.zeros((NCHUNK, NB, 4 * C), f32).at[:, :, 0:3 * C].set(
        Wrc[:, :, perm])
    brow = jnp.zeros((NCHUNK, 1, 4 * C), f32).at[:, 0, 0:3 * C].set(
        brc[:, perm])
    W3 = jnp.zeros((NCHUNK, 3, 4 * C), f32).at[:, :, 3 * C:3 * C + 3].set(
        jnp.broadcast_to(jnp.eye(3, dtype=f32), (NCHUNK, 3, 3)))
    W20f = jnp.moveaxis(W20, 0, 1).reshape(NB, NCHUNK * 4 * C)
    browf = jnp.moveaxis(brow, 0, 1).reshape(1, NCHUNK * 4 * C)
    W3f = jnp.moveaxis(W3, 0, 1).reshape(3, NCHUNK * 4 * C)

    eqt = jnp.stack([node_equi[:, :, c * C:(c + 1) * C].reshape(N, 3 * C)
                     for c in range(NCHUNK)], axis=0)
    init = jnp.stack([
        jnp.concatenate([node_scalar[:, c * C:(c + 1) * C],
                         eqt[c]], axis=1)
        for c in range(NCHUNK)], axis=0)

    t_tab = pl.pallas_call(
        _stage_a_body,
        grid=(NCHUNK, N // BN),
        in_specs=[
            pl.BlockSpec((BN, D), lambda c, i: (i, 0)),
            pl.BlockSpec((D, D), lambda c, i: (0, 0)),
            pl.BlockSpec((D,), lambda c, i: (0,)),
            pl.BlockSpec((1, D, 3 * C), lambda c, i: (c, 0, 0)),
            pl.BlockSpec((1, 1, 3 * C), lambda c, i: (c, 0, 0)),
            pl.BlockSpec((1, BN, 3 * C), lambda c, i: (c, i, 0)),
        ],
        out_specs=pl.BlockSpec((1, BN, 6 * C), lambda c, i: (c, i, 0)),
        out_shape=jax.ShapeDtypeStruct((NCHUNK, N, 6 * C), f32),
    )(node_scalar, W1, b1, W2c, b2c, eqt)

    def stage_b(cols):
        return pl.pallas_call(
            _stage_b_body,
            grid=(E // BE2,),
            in_specs=[
                pl.BlockSpec((BE2, NB), lambda i: (i, 0)),
                pl.BlockSpec((BE2, 1), lambda i: (i, 0)),
                pl.BlockSpec((BE2, 3), lambda i: (i, 0)),
                pl.BlockSpec((NB, 2 * 4 * C), lambda i: (0, 0)),
                pl.BlockSpec((1, 2 * 4 * C), lambda i: (0, 0)),
                pl.BlockSpec((3, 2 * 4 * C), lambda i: (0, 0)),
            ],
            out_specs=pl.BlockSpec((BE2, 2, 4 * C), lambda i: (i, 0, 0)),
            out_shape=jax.ShapeDtypeStruct((E, 2, 4 * C), f32),
        )(rbf, fcut, uvec, W20f[:, cols], browf[:, cols], W3f[:, cols])

    mesh = plsc.VectorSubcoreMesh(core_axis_name="c", subcore_axis_name="s")

    def stage_c(base, f_half, init_half):
        sc = pl.kernel(
            functools.partial(_sc_kernel, base),
            out_type=jax.ShapeDtypeStruct((2, N, 4 * C), f32),
            mesh=mesh,
            scratch_types=[
                pltpu.VMEM_SHARED((N, 4 * C), f32),
                pltpu.VMEM((SBB, BE), jnp.int32),
                pltpu.VMEM((SBB, BE), jnp.int32),
                [pltpu.VMEM((BE, 6 * C), f32) for _ in range(2)],
                [pltpu.VMEM((BE, 4 * C), f32) for _ in range(2)],
                pltpu.VMEM((BE, 4 * C), f32),
                [pltpu.SemaphoreType.DMA for _ in range(2)],
                [pltpu.SemaphoreType.DMA for _ in range(2)],
                [pltpu.SemaphoreType.DMA for _ in range(2)],
            ],
            compiler_params=pltpu.CompilerParams(use_tc_tiling_on_sc=False),
            name=f"sc_chunks_{base}",
        )
        return sc(t_tab, f_half, ctr2, nbr2, init_half)

    ctr2 = edge_index[0].reshape(E // BE, BE)
    nbr2 = edge_index[1].reshape(E // BE, BE)
    half = slice(0, 2 * 4 * C)
    f01 = stage_b(slice(0, 2 * 4 * C))
    out01 = stage_c(0, f01, init[0:2])
    f23 = stage_b(slice(2 * 4 * C, 4 * 4 * C))
    out23 = stage_c(2, f23, init[2:4])
    out = jnp.concatenate([out01, out23], axis=0)

    new_scalar = jnp.moveaxis(out[:, :, 0:C], 0, 1).reshape(N, D)
    new_equi = jnp.transpose(out[:, :, C:].reshape(NCHUNK, N, 3, C),
                             (1, 2, 0, 3)).reshape(N, 3, D)
    return (new_scalar, new_equi)


# speedup vs baseline: 1.6812x; 1.0006x over previous
import functools

import jax
import jax.numpy as jnp
from jax import lax
from jax.experimental import pallas as pl
from jax.experimental.pallas import tpu as pltpu
from jax.experimental.pallas import tpu_sc as plsc

N = 10000
E = 160000
D = 128
NB = 20
C = 32
NCHUNK = 4

BN = 1000
BE2 = 2000
BE = 40
NTILES = 16
EPW = E // NTILES
NPW = 624
NTAIL = N - NPW * NTILES


def _stage_a_body(ns_ref, w1_ref, b1_ref, w2c_ref, b2c_ref, eqt_ref, t_ref):
    ns = ns_ref[...]
    h = ns @ w1_ref[...] + b1_ref[...][None, :]
    h = h * jax.nn.sigmoid(h)
    so = h @ w2c_ref[0] + b2c_ref[0]
    t_ref[0] = jnp.concatenate([so, eqt_ref[0]], axis=1)


def _stage_b_body(rbf_ref, fcut_ref, uv_ref, w20_ref, brow_ref, w3_ref, f_ref):
    fc = fcut_ref[...]
    out = ((rbf_ref[...] * fc) @ w20_ref[...]
           + fc * brow_ref[...]
           + jnp.dot(uv_ref[...], w3_ref[...],
                     precision=jax.lax.Precision.HIGHEST))
    f_ref[...] = out.reshape(out.shape[0], f_ref.shape[1], 4 * C)


ESB = 2000
SBB = ESB // BE
NSB = EPW // ESB
SBPAIR = SBB // 2


def _sc_kernel(base, t_hbm, f_hbm, ctr_hbm, nbr_hbm, init_hbm, out_hbm,
               acc, nbr_sb, ctr_sb, tg_v, f_v, msg_v, gsem, fsem, ssem):
    cid = lax.axis_index("c")
    sid = lax.axis_index("s")
    nstart = sid * NPW

    def compute(slot):
        @plsc.parallel_loop(0, BE, 1, unroll=2)
        def edge(e):
            uvv = f_v[slot][e, pl.ds(3 * C, 16)]
            ub = [jnp.full((16,), uvv[comp], jnp.float32)
                  for comp in range(3)]
            for j in (0, 16):
                tm = tg_v[slot][e, pl.ds(0 + j, 16)]
                te = tg_v[slot][e, pl.ds(C + j, 16)]
                ts = tg_v[slot][e, pl.ds(2 * C + j, 16)]
                a = ts * f_v[slot][e, pl.ds(C + j, 16)]
                b = te * f_v[slot][e, pl.ds(2 * C + j, 16)]
                msg_v[e, pl.ds(0 + j, 16)] = tm * f_v[slot][e, pl.ds(0 + j, 16)]
                for comp in range(3):
                    tx = tg_v[slot][e, pl.ds((3 + comp) * C + j, 16)]
                    msg_v[e, pl.ds((1 + comp) * C + j, 16)] = tx * a + b * ub[comp]

    for local in range(2):
        chunk = base + local
        @pl.when(cid == local)
        def _(chunk=chunk, local=local):
            pltpu.sync_copy(init_hbm.at[local, pl.ds(nstart, NPW)],
                            acc.at[pl.ds(nstart, NPW)])

            @pl.when(sid == NTILES - 1)
            def _():
                pltpu.sync_copy(init_hbm.at[local, pl.ds(NPW * NTILES, NTAIL)],
                                acc.at[pl.ds(NPW * NTILES, NTAIL)])
            plsc.subcore_barrier()

            def superblock(s, carry):
                row0 = sid * (EPW // BE) + s * SBB
                e0s = sid * EPW + s * ESB
                pltpu.sync_copy(nbr_hbm.at[pl.ds(row0, SBB)], nbr_sb)
                pltpu.sync_copy(ctr_hbm.at[pl.ds(row0, SBB)], ctr_sb)

                def load_and_gather(b, slot):
                    pltpu.async_copy(
                        f_hbm.at[pl.ds(e0s + b * BE, BE), local],
                        f_v[slot], fsem[slot])
                    pltpu.async_copy(t_hbm.at[chunk].at[nbr_sb.at[b]],
                                     tg_v[slot], gsem[slot])

                def wait_loads(slot):
                    pltpu.make_async_copy(
                        f_hbm.at[pl.ds(0, BE), local], f_v[slot],
                        fsem[slot]).wait()
                    pltpu.make_async_copy(t_hbm.at[chunk].at[nbr_sb.at[0]],
                                          tg_v[slot], gsem[slot]).wait()

                def do_scatter(b, slot):
                    pltpu.async_copy(msg_v, acc.at[ctr_sb.at[b]], ssem[slot],
                                     add=True)
                    pltpu.make_async_copy(msg_v, acc.at[ctr_sb.at[0]],
                                          ssem[slot]).wait()

                load_and_gather(0, 0)

                def pair(g, carry2):
                    load_and_gather(2 * g + 1, 1)
                    wait_loads(0)
                    compute(0)
                    do_scatter(2 * g, 0)

                    @pl.when(g < SBPAIR - 1)
                    def _():
                        load_and_gather(2 * g + 2, 0)
                    wait_loads(1)
                    compute(1)
                    do_scatter(2 * g + 1, 1)
                    return carry2

                lax.fori_loop(0, SBPAIR, pair, 0)
                return carry

            lax.fori_loop(0, NSB, superblock, 0)
            plsc.subcore_barrier()
            pltpu.sync_copy(acc.at[pl.ds(nstart, NPW)],
                            out_hbm.at[local, pl.ds(nstart, NPW)])

            @pl.when(sid == NTILES - 1)
            def _():
                pltpu.sync_copy(acc.at[pl.ds(NPW * NTILES, NTAIL)],
                                out_hbm.at[local, pl.ds(NPW * NTILES, NTAIL)])
            plsc.subcore_barrier()


def kernel(node_scalar, node_equi, rbf, fcut, uvec, edge_index,
           W1, b1, W2, b2, Wr, br):
    f32 = jnp.float32

    def chunk_cols(w):
        return jnp.stack([
            jnp.concatenate([w[..., k * D + c * C:k * D + c * C + C]
                             for k in range(3)], axis=-1)
            for c in range(NCHUNK)], axis=0)

    W2c = chunk_cols(W2)
    b2c = chunk_cols(b2)[:, None, :]
    Wrc = chunk_cols(Wr)
    brc = chunk_cols(br)
    perm = jnp.concatenate([jnp.arange(C), jnp.arange(2 * C, 3 * C),
                            jnp.arange(C, 2 * C)])
    W20 = jnp.zeros((NCHUNK, NB, 4 * C), f32).at[:, :, 0:3 * C].set(
        Wrc[:, :, perm])
    brow = jnp.zeros((NCHUNK, 1, 4 * C), f32).at[:, 0, 0:3 * C].set(
        brc[:, perm])
    W3 = jnp.zeros((NCHUNK, 3, 4 * C), f32).at[:, :, 3 * C:3 * C + 3].set(
        jnp.broadcast_to(jnp.eye(3, dtype=f32), (NCHUNK, 3, 3)))
    W20f = jnp.moveaxis(W20, 0, 1).reshape(NB, NCHUNK * 4 * C)
    browf = jnp.moveaxis(brow, 0, 1).reshape(1, NCHUNK * 4 * C)
    W3f = jnp.moveaxis(W3, 0, 1).reshape(3, NCHUNK * 4 * C)

    eqt = jnp.stack([node_equi[:, :, c * C:(c + 1) * C].reshape(N, 3 * C)
                     for c in range(NCHUNK)], axis=0)
    init = jnp.stack([
        jnp.concatenate([node_scalar[:, c * C:(c + 1) * C],
                         eqt[c]], axis=1)
        for c in range(NCHUNK)], axis=0)

    t_tab = pl.pallas_call(
        _stage_a_body,
        grid=(NCHUNK, N // BN),
        in_specs=[
            pl.BlockSpec((BN, D), lambda c, i: (i, 0)),
            pl.BlockSpec((D, D), lambda c, i: (0, 0)),
            pl.BlockSpec((D,), lambda c, i: (0,)),
            pl.BlockSpec((1, D, 3 * C), lambda c, i: (c, 0, 0)),
            pl.BlockSpec((1, 1, 3 * C), lambda c, i: (c, 0, 0)),
            pl.BlockSpec((1, BN, 3 * C), lambda c, i: (c, i, 0)),
        ],
        out_specs=pl.BlockSpec((1, BN, 6 * C), lambda c, i: (c, i, 0)),
        out_shape=jax.ShapeDtypeStruct((NCHUNK, N, 6 * C), f32),
    )(node_scalar, W1, b1, W2c, b2c, eqt)

    def stage_b(cols):
        return pl.pallas_call(
            _stage_b_body,
            grid=(E // BE2,),
            in_specs=[
                pl.BlockSpec((BE2, NB), lambda i: (i, 0)),
                pl.BlockSpec((BE2, 1), lambda i: (i, 0)),
                pl.BlockSpec((BE2, 3), lambda i: (i, 0)),
                pl.BlockSpec((NB, 2 * 4 * C), lambda i: (0, 0)),
                pl.BlockSpec((1, 2 * 4 * C), lambda i: (0, 0)),
                pl.BlockSpec((3, 2 * 4 * C), lambda i: (0, 0)),
            ],
            out_specs=pl.BlockSpec((BE2, 2, 4 * C), lambda i: (i, 0, 0)),
            out_shape=jax.ShapeDtypeStruct((E, 2, 4 * C), f32),
        )(rbf, fcut, uvec, W20f[:, cols], browf[:, cols], W3f[:, cols])

    mesh = plsc.VectorSubcoreMesh(core_axis_name="c", subcore_axis_name="s")

    def stage_c(base, f_half, init_half):
        sc = pl.kernel(
            functools.partial(_sc_kernel, base),
            out_type=jax.ShapeDtypeStruct((2, N, 4 * C), f32),
            mesh=mesh,
            scratch_types=[
                pltpu.VMEM_SHARED((N, 4 * C), f32),
                pltpu.VMEM((SBB, BE), jnp.int32),
                pltpu.VMEM((SBB, BE), jnp.int32),
                [pltpu.VMEM((BE, 6 * C), f32) for _ in range(2)],
                [pltpu.VMEM((BE, 4 * C), f32) for _ in range(2)],
                pltpu.VMEM((BE, 4 * C), f32),
                [pltpu.SemaphoreType.DMA for _ in range(2)],
                [pltpu.SemaphoreType.DMA for _ in range(2)],
                [pltpu.SemaphoreType.DMA for _ in range(2)],
            ],
            compiler_params=pltpu.CompilerParams(use_tc_tiling_on_sc=False),
            name=f"sc_chunks_{base}",
        )
        return sc(t_tab, f_half, ctr2, nbr2, init_half)

    ctr2 = edge_index[0].reshape(E // BE, BE)
    nbr2 = edge_index[1].reshape(E // BE, BE)
    f01 = stage_b(slice(0, 2 * 4 * C))
    out01 = stage_c(0, f01, init[0:2])
    f23 = stage_b(slice(2 * 4 * C, 4 * 4 * C))
    out23 = stage_c(2, f23, init[2:4])
    out = jnp.concatenate([out01, out23], axis=0)

    new_scalar = jnp.moveaxis(out[:, :, 0:C], 0, 1).reshape(N, D)
    new_equi = jnp.transpose(out[:, :, C:].reshape(NCHUNK, N, 3, C),
                             (1, 2, 0, 3)).reshape(N, 3, D)
    return (new_scalar, new_equi)
